# Initial kernel scaffold; baseline (speedup 1.0000x reference)
import jax
import jax.numpy as jnp
from jax.experimental import pallas as pl


def kernel(z_a, z_b, z_a_local_features, z_b_local_features, grid_a, grid_b):
    raise NotImplementedError("write your pallas kernel here")


# speedup vs baseline: 2.4015x; 2.4015x over previous
import jax
import jax.numpy as jnp
from jax.experimental import pallas as pl

_B, _N, _C, _D = 16, 1024, 384, 2048
_KA, _KB = 20, 4
_LAMBDA = 25.0
_MU = 25.0
_NU = 1.0
_ALPHA = 0.25
_EPS = 1e-4
_BIG = 3.0e38


def _phase1_kernel(za_ref, zb_ref, ga_ref, gbt_ref,
                   kc_ref, vc_ref, kr_ref, vr_ref):
    za = za_ref[0]
    zb = zb_ref[0]
    g = jax.lax.dot_general(za, zb, (((1,), (1,)), ((), ())),
                            preferred_element_type=jnp.float32,
                            precision=jax.lax.Precision.HIGHEST)
    x2 = jnp.sum(za * za, axis=1, keepdims=True)
    y2 = jnp.sum(zb * zb, axis=1, keepdims=True)
    y2r = jnp.reshape(y2, (1, _N))
    d2 = jnp.maximum(x2 + y2r - 2.0 * g, 0.0)

    rowmin_f = jnp.min(d2, axis=1, keepdims=True)
    colmin_f = jnp.min(d2, axis=0, keepdims=True)

    ga = ga_ref[0]
    gbt = gbt_ref[0]
    dg = ((ga[:, 0:1] - gbt[0:1, :]) ** 2
          + (ga[:, 1:2] - gbt[1:2, :]) ** 2)
    rowmin_g = jnp.min(dg, axis=1, keepdims=True)
    colmin_g = jnp.min(dg, axis=0, keepdims=True)
    iota_j = jax.lax.broadcasted_iota(jnp.int32, (_N, _N), 1)
    iota_i = jax.lax.broadcasted_iota(jnp.int32, (_N, _N), 0)
    row_arg = jnp.min(jnp.where(dg == rowmin_g, iota_j, _N),
                      axis=1, keepdims=True)
    col_arg = jnp.min(jnp.where(dg == colmin_g, iota_i, _N),
                      axis=0, keepdims=True)
    e_a = jnp.sum(jnp.where(iota_j == row_arg, d2, 0.0),
                  axis=1, keepdims=True)
    e_b = jnp.sum(jnp.where(iota_i == col_arg, d2, 0.0),
                  axis=0, keepdims=True)

    kc_ref[0, :, 0:1] = rowmin_f
    kc_ref[0, :, 1:2] = rowmin_g
    vc_ref[0, :, 0:1] = rowmin_f
    vc_ref[0, :, 1:2] = e_a
    kr_ref[0, 0:1, :] = colmin_f
    kr_ref[0, 1:2, :] = colmin_g
    vr_ref[0, 0:1, :] = colmin_f
    vr_ref[0, 1:2, :] = e_b


def _phase2_kernel(keys_ref, vals_ref, za_ref, zb_ref, out_ref):
    keys = keys_ref[:, :, :]
    vals = vals_ref[:, :, :]
    iota_n = jax.lax.broadcasted_iota(jnp.int32, (_B, 4, _N), 2)
    iota_row = jax.lax.broadcasted_iota(jnp.int32, (1, 4, 1), 1)
    krow = jnp.where(iota_row < 3, _KA, _KB)

    def body(t, carry):
        ks, acc = carry
        m = jnp.min(ks, axis=2, keepdims=True)
        idx = jnp.min(jnp.where(ks == m, iota_n, _N),
                      axis=2, keepdims=True)
        onehot = iota_n == idx
        w = (t < krow).astype(jnp.float32)
        acc = acc + jnp.sum(jnp.where(onehot, vals, 0.0),
                            axis=2, keepdims=True) * w
        ks = jnp.where(onehot, _BIG, ks)
        return ks, acc

    acc0 = jnp.zeros((_B, 4, 1), jnp.float32)
    _, acc = jax.lax.fori_loop(0, _KA, body, (keys, acc0))

    lcoef = _LAMBDA * (1.0 - _ALPHA) * 0.5
    coef = jnp.where(iota_row < 3,
                     lcoef / (_B * _KA * _C),
                     lcoef / (_B * _KB * _C))
    local = jnp.sum(acc * coef)

    za = za_ref[:, :]
    zb = zb_ref[:, :]
    inv_g = jnp.mean((za - zb) ** 2)

    def _var_cov(x):
        mu = jnp.mean(x, axis=0, keepdims=True)
        xc = x - mu
        var = jnp.sum(xc * xc, axis=0, keepdims=True) / (_B - 1)
        std = jnp.sqrt(var + _EPS)
        vloss = jnp.mean(jnp.maximum(1.0 - std, 0.0))
        a = jax.lax.dot_general(xc, xc, (((1,), (1,)), ((), ())),
                                preferred_element_type=jnp.float32,
                                precision=jax.lax.Precision.HIGHEST)
        frob = jnp.sum(a * a) / float((_B - 1) ** 2)
        closs = (frob - jnp.sum(var * var)) / _D
        return vloss, closs

    vl_a, cl_a = _var_cov(za)
    vl_b, cl_b = _var_cov(zb)
    global_loss = (_LAMBDA * inv_g + _MU * 0.5 * (vl_a + vl_b)
                   + _NU * (cl_a + cl_b))
    total = _ALPHA * global_loss + local
    out_ref[:, :] = total * jnp.ones((1, 1), jnp.float32)


def kernel(z_a, z_b, z_a_local_features, z_b_local_features, grid_a, grid_b):
    za_l = z_a_local_features.reshape(_B, _N, _C)
    zb_l = z_b_local_features.reshape(_B, _N, _C)
    ga = grid_a.reshape(_B, _N, 2)
    gbt = jnp.swapaxes(grid_b.reshape(_B, _N, 2), 1, 2)

    kc, vc, kr, vr = pl.pallas_call(
        _phase1_kernel,
        grid=(_B,),
        in_specs=[
            pl.BlockSpec((1, _N, _C), lambda b: (b, 0, 0)),
            pl.BlockSpec((1, _N, _C), lambda b: (b, 0, 0)),
            pl.BlockSpec((1, _N, 2), lambda b: (b, 0, 0)),
            pl.BlockSpec((1, 2, _N), lambda b: (b, 0, 0)),
        ],
        out_specs=[
            pl.BlockSpec((1, _N, 2), lambda b: (b, 0, 0)),
            pl.BlockSpec((1, _N, 2), lambda b: (b, 0, 0)),
            pl.BlockSpec((1, 2, _N), lambda b: (b, 0, 0)),
            pl.BlockSpec((1, 2, _N), lambda b: (b, 0, 0)),
        ],
        out_shape=[
            jax.ShapeDtypeStruct((_B, _N, 2), jnp.float32),
            jax.ShapeDtypeStruct((_B, _N, 2), jnp.float32),
            jax.ShapeDtypeStruct((_B, 2, _N), jnp.float32),
            jax.ShapeDtypeStruct((_B, 2, _N), jnp.float32),
        ],
    )(za_l, zb_l, ga, gbt)

    kct = jnp.swapaxes(kc, 1, 2)
    vct = jnp.swapaxes(vc, 1, 2)
    keys = jnp.concatenate(
        [kct[:, 0:1], kr[:, 0:1], kct[:, 1:2], kr[:, 1:2]], axis=1)
    vals = jnp.concatenate(
        [vct[:, 0:1], vr[:, 0:1], vct[:, 1:2], vr[:, 1:2]], axis=1)

    out = pl.pallas_call(
        _phase2_kernel,
        out_shape=jax.ShapeDtypeStruct((1, 1), jnp.float32),
    )(keys, vals, z_a, z_b)
    return out.reshape(())


# speedup vs baseline: 3.7236x; 1.5506x over previous
import jax
import jax.numpy as jnp
from jax.experimental import pallas as pl

_B, _N, _C, _D = 16, 1024, 384, 2048
_KA, _KB = 20, 4
_LAMBDA = 25.0
_MU = 25.0
_NU = 1.0
_ALPHA = 0.25
_EPS = 1e-4
_BIG = 3.0e38


def _phase1_kernel(za_ref, zb_ref, ga_ref, gbt_ref,
                   kc_ref, vc_ref, kr_ref, vr_ref):
    za = za_ref[0]
    zb = zb_ref[0]
    g = jax.lax.dot_general(za, zb, (((1,), (1,)), ((), ())),
                            preferred_element_type=jnp.float32,
                            precision=jax.lax.Precision.DEFAULT)
    x2 = jnp.sum(za * za, axis=1, keepdims=True)
    y2 = jnp.sum(zb * zb, axis=1, keepdims=True)
    y2r = jnp.reshape(y2, (1, _N))
    d2 = jnp.maximum(x2 + y2r - 2.0 * g, 0.0)

    rowmin_f = jnp.min(d2, axis=1, keepdims=True)
    colmin_f = jnp.min(d2, axis=0, keepdims=True)

    ga = ga_ref[0]
    gbt = gbt_ref[0]
    dg = ((ga[:, 0:1] - gbt[0:1, :]) ** 2
          + (ga[:, 1:2] - gbt[1:2, :]) ** 2)
    rowmin_g = jnp.min(dg, axis=1, keepdims=True)
    colmin_g = jnp.min(dg, axis=0, keepdims=True)
    iota_j = jax.lax.broadcasted_iota(jnp.int32, (_N, _N), 1)
    iota_i = jax.lax.broadcasted_iota(jnp.int32, (_N, _N), 0)
    row_arg = jnp.min(jnp.where(dg == rowmin_g, iota_j, _N),
                      axis=1, keepdims=True)
    col_arg = jnp.min(jnp.where(dg == colmin_g, iota_i, _N),
                      axis=0, keepdims=True)
    e_a = jnp.sum(jnp.where(iota_j == row_arg, d2, 0.0),
                  axis=1, keepdims=True)
    e_b = jnp.sum(jnp.where(iota_i == col_arg, d2, 0.0),
                  axis=0, keepdims=True)

    kc_ref[0, :, 0:1] = rowmin_f
    kc_ref[0, :, 1:2] = rowmin_g
    vc_ref[0, :, 0:1] = rowmin_f
    vc_ref[0, :, 1:2] = e_a
    kr_ref[0, 0:1, :] = colmin_f
    kr_ref[0, 1:2, :] = colmin_g
    vr_ref[0, 0:1, :] = colmin_f
    vr_ref[0, 1:2, :] = e_b


def _phase2_kernel(keys_ref, vals_ref, za_ref, zb_ref, out_ref):
    keys = keys_ref[:, :, :]
    vals = vals_ref[:, :, :]
    iota_n = jax.lax.broadcasted_iota(jnp.int32, (_B, 4, _N), 2)
    iota_row = jax.lax.broadcasted_iota(jnp.int32, (1, 4, 1), 1)
    krow = jnp.where(iota_row < 3, _KA, _KB)

    def body(t, carry):
        ks, acc = carry
        m = jnp.min(ks, axis=2, keepdims=True)
        idx = jnp.min(jnp.where(ks == m, iota_n, _N),
                      axis=2, keepdims=True)
        onehot = iota_n == idx
        w = (t < krow).astype(jnp.float32)
        acc = acc + jnp.sum(jnp.where(onehot, vals, 0.0),
                            axis=2, keepdims=True) * w
        ks = jnp.where(onehot, _BIG, ks)
        return ks, acc

    acc0 = jnp.zeros((_B, 4, 1), jnp.float32)
    _, acc = jax.lax.fori_loop(0, _KA, body, (keys, acc0))

    lcoef = _LAMBDA * (1.0 - _ALPHA) * 0.5
    coef = jnp.where(iota_row < 3,
                     lcoef / (_B * _KA * _C),
                     lcoef / (_B * _KB * _C))
    local = jnp.sum(acc * coef)

    za = za_ref[:, :]
    zb = zb_ref[:, :]
    inv_g = jnp.mean((za - zb) ** 2)

    def _var_cov(x):
        mu = jnp.mean(x, axis=0, keepdims=True)
        xc = x - mu
        var = jnp.sum(xc * xc, axis=0, keepdims=True) / (_B - 1)
        std = jnp.sqrt(var + _EPS)
        vloss = jnp.mean(jnp.maximum(1.0 - std, 0.0))
        a = jax.lax.dot_general(xc, xc, (((1,), (1,)), ((), ())),
                                preferred_element_type=jnp.float32,
                                precision=jax.lax.Precision.HIGHEST)
        frob = jnp.sum(a * a) / float((_B - 1) ** 2)
        closs = (frob - jnp.sum(var * var)) / _D
        return vloss, closs

    vl_a, cl_a = _var_cov(za)
    vl_b, cl_b = _var_cov(zb)
    global_loss = (_LAMBDA * inv_g + _MU * 0.5 * (vl_a + vl_b)
                   + _NU * (cl_a + cl_b))
    total = _ALPHA * global_loss + local
    out_ref[:, :] = total * jnp.ones((1, 1), jnp.float32)


def kernel(z_a, z_b, z_a_local_features, z_b_local_features, grid_a, grid_b):
    za_l = z_a_local_features.reshape(_B, _N, _C)
    zb_l = z_b_local_features.reshape(_B, _N, _C)
    ga = grid_a.reshape(_B, _N, 2)
    gbt = jnp.swapaxes(grid_b.reshape(_B, _N, 2), 1, 2)

    kc, vc, kr, vr = pl.pallas_call(
        _phase1_kernel,
        grid=(_B,),
        in_specs=[
            pl.BlockSpec((1, _N, _C), lambda b: (b, 0, 0)),
            pl.BlockSpec((1, _N, _C), lambda b: (b, 0, 0)),
            pl.BlockSpec((1, _N, 2), lambda b: (b, 0, 0)),
            pl.BlockSpec((1, 2, _N), lambda b: (b, 0, 0)),
        ],
        out_specs=[
            pl.BlockSpec((1, _N, 2), lambda b: (b, 0, 0)),
            pl.BlockSpec((1, _N, 2), lambda b: (b, 0, 0)),
            pl.BlockSpec((1, 2, _N), lambda b: (b, 0, 0)),
            pl.BlockSpec((1, 2, _N), lambda b: (b, 0, 0)),
        ],
        out_shape=[
            jax.ShapeDtypeStruct((_B, _N, 2), jnp.float32),
            jax.ShapeDtypeStruct((_B, _N, 2), jnp.float32),
            jax.ShapeDtypeStruct((_B, 2, _N), jnp.float32),
            jax.ShapeDtypeStruct((_B, 2, _N), jnp.float32),
        ],
    )(za_l, zb_l, ga, gbt)

    kct = jnp.swapaxes(kc, 1, 2)
    vct = jnp.swapaxes(vc, 1, 2)
    keys = jnp.concatenate(
        [kct[:, 0:1], kr[:, 0:1], kct[:, 1:2], kr[:, 1:2]], axis=1)
    vals = jnp.concatenate(
        [vct[:, 0:1], vr[:, 0:1], vct[:, 1:2], vr[:, 1:2]], axis=1)

    out = pl.pallas_call(
        _phase2_kernel,
        out_shape=jax.ShapeDtypeStruct((1, 1), jnp.float32),
    )(keys, vals, z_a, z_b)
    return out.reshape(())
